# baseline (speedup 1.0000x reference)
import functools

import jax
import jax.numpy as jnp
from jax import lax
from jax.experimental import pallas as pl
from jax.experimental.pallas import tpu as pltpu
from jax.experimental.pallas import tpu_sc as plsc

_B = 4096 * 26
_D = 64
_L = 128
_NW = 32
_ROWS_PER_W = _B // _NW
_HALF_ROWS = _ROWS_PER_W // 2
_HALF_GROUPS = _HALF_ROWS // _L


def _emb_body(idx_hbm, table_hbm, out_hbm, idx_v, rows_v, sem):
    wid = lax.axis_index("s") * 2 + lax.axis_index("c")
    base = wid * _ROWS_PER_W
    for half in range(2):
        r0 = base + half * _HALF_ROWS
        pltpu.sync_copy(idx_hbm.at[pl.ds(r0, _HALF_ROWS)], idx_v)
        copies = [
            pltpu.async_copy(
                table_hbm.at[idx_v.at[pl.ds(g * _L, _L)]],
                rows_v.at[pl.ds(g * _L, _L)],
                sem,
            )
            for g in range(_HALF_GROUPS)
        ]
        for c in copies:
            c.wait()

        def _mask_fix(i, carry):
            r = i * 16
            for j in range(16):
                ij = plsc.load_gather(
                    idx_v, [jnp.full((16,), r + j, jnp.int32)]
                )
                mj = jnp.where(ij == 0, 0.0, 1.0).astype(jnp.float32)
                for q in range(4):
                    sl = (r + j, pl.ds(q * 16, 16))
                    rows_v[sl] = rows_v[sl] * mj
            return carry

        lax.fori_loop(0, _HALF_ROWS // 16, _mask_fix, 0)
        pltpu.sync_copy(rows_v, out_hbm.at[pl.ds(r0, _HALF_ROWS)])


_emb = functools.partial(
    pl.kernel,
    out_type=jax.ShapeDtypeStruct((_B, _D), jnp.float32),
    mesh=plsc.VectorSubcoreMesh(core_axis_name="c", subcore_axis_name="s"),
    compiler_params=pltpu.CompilerParams(
        needs_layout_passes=False, use_tc_tiling_on_sc=False
    ),
    scratch_types=[
        pltpu.VMEM((_HALF_ROWS,), jnp.int32),
        pltpu.VMEM((_HALF_ROWS, _D), jnp.float32),
        pltpu.SemaphoreType.DMA,
    ],
)(_emb_body)


def kernel(input, table):
    idx_flat = input.reshape(_B)
    out = _emb(idx_flat, table)
    return out.reshape(input.shape[0], input.shape[1], _D)


# speedup vs baseline: 1.0809x; 1.0809x over previous
import functools

import jax
import jax.numpy as jnp
from jax import lax
from jax.experimental import pallas as pl
from jax.experimental.pallas import tpu as pltpu
from jax.experimental.pallas import tpu_sc as plsc

_B = 4096 * 26
_D = 64
_NW = 32
_ROWS_PER_W = _B // _NW
_NCHUNK = 4
_CHUNK = _ROWS_PER_W // _NCHUNK
_NG = 8
_GL = _CHUNK // _NG


def _emb_body(idx_hbm, table_hbm, out_hbm, idx_v, rows0, rows1, g0s, g1s,
              w0s, w1s):
    wid = lax.axis_index("s") * 2 + lax.axis_index("c")
    base = wid * _ROWS_PER_W
    pltpu.sync_copy(idx_hbm.at[pl.ds(base, _ROWS_PER_W)], idx_v)
    bufs = (rows0, rows1)
    gsems = (g0s, g1s)
    wsems = (w0s, w1s)

    def fire(c):
        b = c & 1
        return [
            pltpu.async_copy(
                table_hbm.at[idx_v.at[pl.ds(c * _CHUNK + g * _GL, _GL)]],
                bufs[b].at[pl.ds(g * _GL, _GL)],
                gsems[b],
            )
            for g in range(_NG)
        ]

    writes = [None] * _NCHUNK
    gathers = [None] * _NCHUNK
    gathers[0] = fire(0)
    for c in range(_NCHUNK):
        b = c & 1
        if c + 1 < _NCHUNK:
            if c >= 1:
                writes[c - 1].wait()
            gathers[c + 1] = fire(c + 1)
        for cp in gathers[c]:
            cp.wait()

        def _mask_fix(i, carry):
            r = i * 16
            idxs = idx_v[pl.ds(c * _CHUNK + r, 16)]

            @pl.when(jnp.any(idxs == 0))
            def _():
                for j in range(16):
                    ij = plsc.load_gather(
                        idx_v, [jnp.full((16,), c * _CHUNK + r + j, jnp.int32)]
                    )
                    mj = jnp.where(ij == 0, 0.0, 1.0).astype(jnp.float32)
                    for q in range(4):
                        sl = (r + j, pl.ds(q * 16, 16))
                        bufs[b][sl] = bufs[b][sl] * mj

            return carry

        lax.fori_loop(0, _CHUNK // 16, _mask_fix, 0)
        writes[c] = pltpu.async_copy(
            bufs[b], out_hbm.at[pl.ds(base + c * _CHUNK, _CHUNK)], wsems[b]
        )
    writes[_NCHUNK - 2].wait()
    writes[_NCHUNK - 1].wait()


_emb = functools.partial(
    pl.kernel,
    out_type=jax.ShapeDtypeStruct((_B, _D), jnp.float32),
    mesh=plsc.VectorSubcoreMesh(core_axis_name="c", subcore_axis_name="s"),
    compiler_params=pltpu.CompilerParams(
        needs_layout_passes=False, use_tc_tiling_on_sc=False
    ),
    scratch_types=[
        pltpu.VMEM((_ROWS_PER_W,), jnp.int32),
        pltpu.VMEM((_CHUNK, _D), jnp.float32),
        pltpu.VMEM((_CHUNK, _D), jnp.float32),
        pltpu.SemaphoreType.DMA,
        pltpu.SemaphoreType.DMA,
        pltpu.SemaphoreType.DMA,
        pltpu.SemaphoreType.DMA,
    ],
)(_emb_body)


def kernel(input, table):
    idx_flat = input.reshape(_B)
    out = _emb(idx_flat, table)
    return out.reshape(input.shape[0], input.shape[1], _D)
